# Initial kernel scaffold; baseline (speedup 1.0000x reference)
import jax
import jax.numpy as jnp
from jax.experimental import pallas as pl


def kernel(x, edge_index, Wt1, bt1, Wt2, bt2, Wp1, bp1, Wp2, bp2):
    raise NotImplementedError("write your pallas kernel here")


# speedup vs baseline: 11.0804x; 11.0804x over previous
import functools

import jax
import jax.numpy as jnp
from jax import lax
from jax.experimental import pallas as pl
from jax.experimental.pallas import tpu as pltpu
from jax.experimental.pallas import tpu_sc as plsc

N = 10000
E = 320000
D_IN = 128
D_H = 128
D_OUT = 64
D = 128

NC = 2
NS = 16
NW = NC * NS
EPW = E // NW
CHUNK = 80
NCHUNK = EPW // CHUNK
NPAD = 10240
ROWS_PER_TILE = NPAD // NS


def _make_sc_aggregate(count_deg):
    mesh = plsc.VectorSubcoreMesh(core_axis_name="c", subcore_axis_name="s")
    out_type = [jax.ShapeDtypeStruct((NC, NPAD, D), jnp.float32)]
    if count_deg:
        out_type.append(jax.ShapeDtypeStruct((NW, NPAD), jnp.float32))

    @functools.partial(
        pl.kernel,
        mesh=mesh,
        out_type=out_type,
        compiler_params=pltpu.CompilerParams(use_tc_tiling_on_sc=False,
                                             needs_layout_passes=False),
        scratch_types=[
            pltpu.VMEM((NCHUNK, CHUNK), jnp.int32),
            pltpu.VMEM((NCHUNK, CHUNK), jnp.int32),
            pltpu.VMEM((CHUNK, D), jnp.float32),
            pltpu.VMEM((NPAD,), jnp.float32),
            pltpu.VMEM_SHARED((NPAD, D), jnp.float32),
            pltpu.SemaphoreType.DMA,
        ],
    )
    def agg(feat_hbm, src_hbm, dst_hbm, zeros_hbm, *rest):
        if count_deg:
            out_hbm, deg_hbm, src_v, dst_v, buf_v, deg_v, acc_sh, sem = rest
        else:
            out_hbm, src_v, dst_v, buf_v, deg_v, acc_sh, sem = rest
        c = lax.axis_index("c")
        s = lax.axis_index("s")
        b = c * NS + s
        r0 = s * ROWS_PER_TILE
        pltpu.sync_copy(zeros_hbm.at[pl.ds(r0, ROWS_PER_TILE)],
                        acc_sh.at[pl.ds(r0, ROWS_PER_TILE)])
        pltpu.sync_copy(src_hbm.at[b], src_v)
        pltpu.sync_copy(dst_hbm.at[b], dst_v)
        if count_deg:
            def zero_deg(t, carry):
                deg_v[pl.ds(t * 16, 16)] = jnp.zeros((16,), jnp.float32)
                return carry
            lax.fori_loop(0, NPAD // 16, zero_deg, 0)
        plsc.subcore_barrier()

        def chunk(j, carry):
            gather = pltpu.async_copy(feat_hbm.at[src_v.at[j]], buf_v, sem)
            if count_deg:
                def upd(k, carry2):
                    dv = dst_v[j, pl.ds(k * 16, 16)]
                    plsc.addupdate_scatter(deg_v, [dv],
                                           jnp.ones((16,), jnp.float32))
                    return carry2
                lax.fori_loop(0, CHUNK // 16, upd, 0)
            gather.wait()
            pltpu.sync_copy(buf_v, acc_sh.at[dst_v.at[j]], add=True)
            return carry

        lax.fori_loop(0, NCHUNK, chunk, 0)
        plsc.subcore_barrier()
        pltpu.sync_copy(acc_sh.at[pl.ds(r0, ROWS_PER_TILE)],
                        out_hbm.at[c, pl.ds(r0, ROWS_PER_TILE)])
        if count_deg:
            pltpu.sync_copy(deg_v, deg_hbm.at[b])

    return agg


_BN = 1000
_BN1 = 1024


def _tc_forward(p, deg, Wt1, bt1, Wp1, bp1, Wt2, Wp2):
    def body(p_ref, deg_ref, wt1_ref, bt1_ref, wp1_ref, bp1_ref, wt2_ref,
             wp2_ref, z_ref, inv_ref):
        degs = jnp.sum(deg_ref[...], axis=0)[:, None]
        inv = 1.0 / jnp.maximum(degs, 1.0)
        a = (p_ref[0] + p_ref[1]) * inv
        ht = jnp.maximum(
            jnp.dot(a, wt1_ref[...], preferred_element_type=jnp.float32)
            + bt1_ref[...], 0.0)
        hp = jnp.maximum(
            jnp.dot(a, wp1_ref[...], preferred_element_type=jnp.float32)
            + bp1_ref[...], 0.0)
        z_ref[:, :D_OUT] = (
            jnp.dot(hp, wp2_ref[...], preferred_element_type=jnp.float32)
            - jnp.dot(ht, wt2_ref[...], preferred_element_type=jnp.float32))
        z_ref[:, D_OUT:] = jnp.zeros((_BN1, D - D_OUT), jnp.float32)
        inv_ref[...] = inv

    return pl.pallas_call(
        body,
        grid=(NPAD // _BN1,),
        in_specs=[
            pl.BlockSpec((NC, _BN1, D), lambda i: (0, i, 0)),
            pl.BlockSpec((NW, _BN1), lambda i: (0, i)),
            pl.BlockSpec((D_IN, D_H), lambda i: (0, 0)),
            pl.BlockSpec((1, D_H), lambda i: (0, 0)),
            pl.BlockSpec((D_IN, D_H), lambda i: (0, 0)),
            pl.BlockSpec((1, D_H), lambda i: (0, 0)),
            pl.BlockSpec((D_H, D_OUT), lambda i: (0, 0)),
            pl.BlockSpec((D_H, D_OUT), lambda i: (0, 0)),
        ],
        out_specs=[
            pl.BlockSpec((_BN1, D), lambda i: (i, 0)),
            pl.BlockSpec((_BN1, 1), lambda i: (i, 0)),
        ],
        out_shape=[
            jax.ShapeDtypeStruct((NPAD, D), jnp.float32),
            jax.ShapeDtypeStruct((NPAD, 1), jnp.float32),
        ],
    )(p, deg, Wt1, bt1, Wp1, bp1, Wt2, Wp2)


def _tc_loss(q, invd, db):
    grid_n = N // _BN

    def body(q_ref, inv_ref, db_ref, out_ref):
        i = pl.program_id(0)
        diff = (q_ref[0] + q_ref[1]) * inv_ref[...] + db_ref[...]
        ssq = jnp.sum(diff * diff)
        prev = jnp.where(i == 0, 0.0, out_ref[0, 0])
        tot = prev + ssq
        out_ref[0, 0] = jnp.where(i == grid_n - 1,
                                  tot * (1.0 / (N * D_OUT)), tot)

    return pl.pallas_call(
        body,
        grid=(grid_n,),
        in_specs=[
            pl.BlockSpec((NC, _BN, D), lambda i: (0, i, 0)),
            pl.BlockSpec((_BN, 1), lambda i: (i, 0)),
            pl.BlockSpec((1, D), lambda i: (0, 0)),
        ],
        out_specs=pl.BlockSpec(memory_space=pltpu.SMEM),
        out_shape=jax.ShapeDtypeStruct((1, 1), jnp.float32),
    )(q, invd, db)


def kernel(x, edge_index, Wt1, bt1, Wt2, bt2, Wp1, bp1, Wp2, bp2):
    src = edge_index[0].reshape(NW, NCHUNK, CHUNK)
    dst = edge_index[1].reshape(NW, NCHUNK, CHUNK)
    zeros = jnp.zeros((NPAD, D), jnp.float32)

    p1, deg = _make_sc_aggregate(True)(x, src, dst, zeros)
    z, invd = _tc_forward(p1, deg, Wt1, bt1.reshape(1, D_H), Wp1,
                          bp1.reshape(1, D_H), Wt2, Wp2)
    (p2,) = _make_sc_aggregate(False)(z, src, dst, zeros)
    db = jnp.concatenate(
        [bp2 - bt2, jnp.zeros((D - D_OUT,), jnp.float32)]).reshape(1, D)
    loss = _tc_loss(p2, invd, db)
    return loss[0, 0]
